# speedup vs baseline: 8.1687x; 8.1687x over previous
import functools
import math

import jax
import jax.numpy as jnp
from jax import lax
from jax.experimental import pallas as pl
from jax.experimental.pallas import tpu as pltpu
from jax.experimental.pallas import tpu_sc as plsc

N_NODES = 10000
K = 27
E_PER_K = 12000
E = K * E_PER_K
D = 128

SCALE = math.sqrt(2.0) / math.sqrt(float(K * D))

NC = 2
NS = 16
NW = NC * NS
B = 128
NB = 80
E_W = NB * B
E_PAD = NW * E_W
ACC_ROWS = 10240
ROWS_PER_TILE = ACC_ROWS // NS


def _z_body(x_ref, w_ref, z_ref):
    z_ref[0] = jnp.dot(x_ref[...], w_ref[0] * SCALE,
                       preferred_element_type=jnp.float32)


def _compute_z(x, W):
    return pl.pallas_call(
        _z_body,
        grid=(K,),
        in_specs=[
            pl.BlockSpec((N_NODES, D), lambda k: (0, 0)),
            pl.BlockSpec((1, D, D), lambda k: (k, 0, 0)),
        ],
        out_specs=pl.BlockSpec((1, N_NODES, D), lambda k: (k, 0, 0)),
        out_shape=jax.ShapeDtypeStruct((K, N_NODES, D), jnp.float32),
    )(x, W)


_mesh = plsc.VectorSubcoreMesh(core_axis_name="c", subcore_axis_name="s")


@functools.partial(
    pl.kernel,
    mesh=_mesh,
    out_type=jax.ShapeDtypeStruct((NC, ACC_ROWS, D), jnp.float32),
    scratch_types=[
        pltpu.VMEM((NB, B), jnp.int32),
        pltpu.VMEM((NB, B), jnp.int32),
        pltpu.VMEM((B, D), jnp.float32),
        pltpu.VMEM_SHARED((ACC_ROWS, D), jnp.float32),
        pltpu.SemaphoreType.DMA,
    ],
)
def _sc_scatter(z_hbm, gidx_hbm, dst_hbm, out_hbm, idx_v, dst_v, rows_v,
                acc_sh, sem):
    c = lax.axis_index("c")
    s = lax.axis_index("s")
    wid = c * NS + s

    def _zero_row(r, carry):
        for cc in range(D // 16):
            rows_v[r, pl.ds(cc * 16, 16)] = jnp.zeros((16,), jnp.float32)
        return carry

    lax.fori_loop(0, B, _zero_row, 0)
    for i in range(ROWS_PER_TILE // B):
        pltpu.sync_copy(rows_v,
                        acc_sh.at[pl.ds(s * ROWS_PER_TILE + i * B, B)])
    plsc.subcore_barrier()

    pltpu.sync_copy(gidx_hbm.at[wid], idx_v)
    pltpu.sync_copy(dst_hbm.at[wid], dst_v)

    def _step(j, carry):
        pltpu.async_copy(z_hbm.at[idx_v.at[j]], rows_v, sem).wait()
        pltpu.sync_copy(rows_v, acc_sh.at[dst_v.at[j]], add=True)
        return carry

    lax.fori_loop(0, NB, _step, 0)
    plsc.subcore_barrier()

    pltpu.sync_copy(acc_sh.at[pl.ds(s * ROWS_PER_TILE, ROWS_PER_TILE)],
                    out_hbm.at[c, pl.ds(s * ROWS_PER_TILE, ROWS_PER_TILE)])


def _merge_body(p_ref, b_ref, o_ref):
    o_ref[...] = p_ref[0] + p_ref[1] + b_ref[...]


def _merge(partial, bias2d):
    blk = 2000
    return pl.pallas_call(
        _merge_body,
        grid=(N_NODES // blk,),
        in_specs=[
            pl.BlockSpec((NC, blk, D), lambda i: (0, i, 0)),
            pl.BlockSpec((1, D), lambda i: (0, 0)),
        ],
        out_specs=pl.BlockSpec((blk, D), lambda i: (i, 0)),
        out_shape=jax.ShapeDtypeStruct((N_NODES, D), jnp.float32),
    )(partial, bias2d)


def kernel(x, edge_index, W, bias):
    src = edge_index[0]
    dst = edge_index[1]
    koff = (jnp.arange(E, dtype=jnp.int32) // E_PER_K) * N_NODES
    gidx = src + koff
    npad = E_PAD - E
    pad_g = jnp.arange(npad, dtype=jnp.int32) % (K * N_NODES)
    pad_d = N_NODES + jnp.arange(npad, dtype=jnp.int32) % (ACC_ROWS - N_NODES)
    gidx_p = jnp.concatenate([gidx, pad_g]).reshape(NW, NB, B)
    dst_p = jnp.concatenate([dst, pad_d]).reshape(NW, NB, B)

    z = _compute_z(x, W).reshape(K * N_NODES, D)
    partial = _sc_scatter(z, gidx_p, dst_p)
    return _merge(partial, bias.reshape(1, D))


# speedup vs baseline: 10.8946x; 1.3337x over previous
import functools
import math

import jax
import jax.numpy as jnp
from jax import lax
from jax.experimental import pallas as pl
from jax.experimental.pallas import tpu as pltpu
from jax.experimental.pallas import tpu_sc as plsc

N_NODES = 10000
K = 27
E_PER_K = 12000
E = K * E_PER_K
D = 128

SCALE = math.sqrt(2.0) / math.sqrt(float(K * D))

NC = 2
NS = 16
NW = NC * NS
B = 128
NB = 80
PH = 2
NBP = NB // PH
NBPI = NBP + 1
E_W = NB * B
E_PAD = NW * E_W
ACC_ROWS = 10240
ROWS_PER_TILE = ACC_ROWS // NS


def _z_body(x_ref, w_ref, z_ref):
    z_ref[0] = jnp.dot(x_ref[...], w_ref[0] * SCALE,
                       preferred_element_type=jnp.float32)


def _compute_z(x, W):
    return pl.pallas_call(
        _z_body,
        grid=(K,),
        in_specs=[
            pl.BlockSpec((N_NODES, D), lambda k: (0, 0)),
            pl.BlockSpec((1, D, D), lambda k: (k, 0, 0)),
        ],
        out_specs=pl.BlockSpec((1, N_NODES, D), lambda k: (k, 0, 0)),
        out_shape=jax.ShapeDtypeStruct((K, N_NODES, D), jnp.float32),
    )(x, W)


_mesh = plsc.VectorSubcoreMesh(core_axis_name="c", subcore_axis_name="s")


@functools.partial(
    pl.kernel,
    mesh=_mesh,
    out_type=jax.ShapeDtypeStruct((NC, ACC_ROWS, D), jnp.float32),
    scratch_types=[
        pltpu.VMEM((NBPI, B), jnp.int32),
        pltpu.VMEM((NBP, B), jnp.int32),
        pltpu.VMEM((B, D), jnp.float32),
        pltpu.VMEM((B, D), jnp.float32),
        pltpu.VMEM_SHARED((ACC_ROWS, D), jnp.float32),
        pltpu.SemaphoreType.DMA,
        pltpu.SemaphoreType.DMA,
    ],
)
def _sc_scatter(z_hbm, gidx_hbm, dst_hbm, out_hbm, idx_v, dst_v, rows0,
                rows1, acc_sh, sem0, sem1):
    c = lax.axis_index("c")
    s = lax.axis_index("s")
    wid = c * NS + s

    def _zero_row(r, carry):
        for cc in range(D // 16):
            rows0[r, pl.ds(cc * 16, 16)] = jnp.zeros((16,), jnp.float32)
        return carry

    lax.fori_loop(0, B, _zero_row, 0)

    def _zero_chunk(i, carry):
        pltpu.sync_copy(rows0,
                        acc_sh.at[pl.ds(s * ROWS_PER_TILE + i * B, B)])
        return carry

    lax.fori_loop(0, ROWS_PER_TILE // B, _zero_chunk, 0)
    plsc.subcore_barrier()

    def _fire(j, buf, sem):
        pltpu.async_copy(z_hbm.at[idx_v.at[j]], buf, sem)

    def _wait(j, buf, sem):
        pltpu.make_async_copy(z_hbm.at[idx_v.at[j]], buf, sem).wait()

    def _scat(j, buf):
        pltpu.sync_copy(buf, acc_sh.at[dst_v.at[j]], add=True)

    for p in range(PH):
        pltpu.sync_copy(gidx_hbm.at[wid, p], idx_v)
        pltpu.sync_copy(dst_hbm.at[wid, p], dst_v)
        _fire(0, rows0, sem0)

        def _pair(g, carry):
            j = 2 * g
            _fire(j + 1, rows1, sem1)
            _wait(j, rows0, sem0)
            _scat(j, rows0)
            _fire(j + 2, rows0, sem0)
            _wait(j + 1, rows1, sem1)
            _scat(j + 1, rows1)
            return carry

        lax.fori_loop(0, NBP // 2, _pair, 0)
        _wait(NBP, rows0, sem0)
    plsc.subcore_barrier()

    pltpu.sync_copy(acc_sh.at[pl.ds(s * ROWS_PER_TILE, ROWS_PER_TILE)],
                    out_hbm.at[c, pl.ds(s * ROWS_PER_TILE, ROWS_PER_TILE)])


def _merge_body(p_ref, b_ref, o_ref):
    o_ref[...] = p_ref[0] + p_ref[1] + b_ref[...]


def _merge(partial, bias2d):
    blk = 2000
    return pl.pallas_call(
        _merge_body,
        grid=(N_NODES // blk,),
        in_specs=[
            pl.BlockSpec((NC, blk, D), lambda i: (0, i, 0)),
            pl.BlockSpec((1, D), lambda i: (0, 0)),
        ],
        out_specs=pl.BlockSpec((blk, D), lambda i: (i, 0)),
        out_shape=jax.ShapeDtypeStruct((N_NODES, D), jnp.float32),
    )(partial, bias2d)


def kernel(x, edge_index, W, bias):
    src = edge_index[0]
    dst = edge_index[1]
    koff = (jnp.arange(E, dtype=jnp.int32) // E_PER_K) * N_NODES
    gidx = src + koff
    npad = E_PAD - E
    pad_g = jnp.arange(npad, dtype=jnp.int32) % (K * N_NODES)
    pad_d = N_NODES + jnp.arange(npad, dtype=jnp.int32) % (ACC_ROWS - N_NODES)
    gidx_p = jnp.concatenate([gidx, pad_g]).reshape(NW, PH, NBP, B)
    dst_p = jnp.concatenate([dst, pad_d]).reshape(NW, PH, NBP, B)
    dummy = (jnp.arange(NW * PH * B, dtype=jnp.int32)
             .reshape(NW, PH, 1, B) % (K * N_NODES))
    gidx_p = jnp.concatenate([gidx_p, dummy], axis=2)

    z = _compute_z(x, W).reshape(K * N_NODES, D)
    partial = _sc_scatter(z, gidx_p, dst_p)
    return _merge(partial, bias.reshape(1, D))
